# baseline (speedup 1.0000x reference)
import functools

import jax
import jax.numpy as jnp
from jax import lax
from jax.experimental import pallas as pl
from jax.experimental.pallas import tpu as pltpu
from jax.experimental.pallas import tpu_sc as plsc

B = 4
N = 10000
E = 320000
DIN = 128
DOUT = 128

NPAD = 10240
ROWS_PER_TILE = NPAD // 16
CHUNK = 128
E_DEG = 327680
DEG_CHUNKS = E_DEG // 32 // CHUNK

LCHUNK = 16
NB = 6
SLAB = 4096
SLAB_PER_TILE = SLAB // 16
NPASS = 3
UPAD = SLAB * NPASS
NAGG = 10112
AGG_PER_TILE = NAGG // 16
DUMP0 = 10000
NH = 254
NS = 508
NV = NH + NS
EH_T = NPASS * NH * LCHUNK
ES_T = NS * LCHUNK
EPT = EH_T + ES_T
E_LAP = EPT * 16

@functools.cache
def _mesh():
    return plsc.VectorSubcoreMesh(core_axis_name="c", subcore_axis_name="s")


def _sc_deg_body(dsts, ones_pat, zeros_slab, deg_out, dst_v, ones_v, deg_sh,
                 sem):
    c = lax.axis_index("c")
    s = lax.axis_index("s")
    pltpu.sync_copy(dsts.at[c].at[s], dst_v)
    pltpu.sync_copy(ones_pat, ones_v)
    pltpu.sync_copy(
        zeros_slab, deg_sh.at[pl.ds(s * ROWS_PER_TILE, ROWS_PER_TILE)])
    plsc.subcore_barrier()

    @pl.loop(0, DEG_CHUNKS)
    def _(j):
        pltpu.sync_copy(ones_v, deg_sh.at[dst_v.at[j]], add=True)

    plsc.subcore_barrier()
    pltpu.sync_copy(
        deg_sh.at[pl.ds(s * ROWS_PER_TILE, ROWS_PER_TILE)],
        deg_out.at[c].at[pl.ds(s * ROWS_PER_TILE, ROWS_PER_TILE)],
    )


@jax.jit
def _sc_deg(dsts32, ones_pat, zeros_slab):
    return pl.kernel(
        _sc_deg_body,
        out_type=jax.ShapeDtypeStruct((2, NPAD, DIN), jnp.float32),
        mesh=_mesh(),
        scratch_types=[
            pltpu.VMEM((DEG_CHUNKS, CHUNK), jnp.int32),
            pltpu.VMEM((CHUNK, DIN), jnp.float32),
            pltpu.VMEM_SHARED((NPAD, DIN), jnp.float32),
            pltpu.SemaphoreType.DMA,
        ],
    )(dsts32, ones_pat, zeros_slab)


def _lap_pass(table, edges3, b, p, s, ibufs, bufs, slab_sh, agg_sh,
              semi, semg, semsc):
    pltpu.sync_copy(
        table.at[b].at[pl.ds(p * SLAB + s * SLAB_PER_TILE, SLAB_PER_TILE)],
        slab_sh.at[pl.ds(s * SLAB_PER_TILE, SLAB_PER_TILE)],
    )
    plsc.subcore_barrier()

    eds = edges3.at[p].at[s]
    tab = table.at[b]

    def wait_idx(k):
        pltpu.make_async_copy(eds.at[0], ibufs[k], semi[k]).wait()

    def fire_gather(k, is_h):
        src = tab if is_h else slab_sh
        pltpu.async_copy(src.at[ibufs[k].at[0]], bufs[k], semg[k])

    def wait_gather(k):
        pltpu.make_async_copy(slab_sh.at[ibufs[k].at[0]], bufs[k],
                              semg[k]).wait()

    def fire_scatter(k):
        pltpu.async_copy(bufs[k], agg_sh.at[ibufs[k].at[1]], semsc[k],
                         add=True)

    def wait_scatter(k):
        pltpu.make_async_copy(bufs[k], agg_sh.at[ibufs[k].at[1]],
                              semsc[k]).wait()

    for k in range(3):
        pltpu.async_copy(eds.at[k], ibufs[k], semi[k])
    for k in range(2):
        wait_idx(k)
        fire_gather(k, k % 3 == 0)

    @pl.loop(0, NV // NB)
    def _(gi):
        base = NB * gi
        for k in range(NB):
            v = base + k
            k2 = (k + 2) % NB
            k3 = (k + 3) % NB
            wait_gather(k)
            fire_scatter(k)

            @pl.when(v >= 3)
            def _():
                wait_scatter(k3)

            @pl.when(v + 3 < NV)
            def _():
                pltpu.async_copy(eds.at[v + 3], ibufs[k3], semi[k3])

            @pl.when(v + 2 < NV)
            def _():
                wait_idx(k2)
                fire_gather(k2, (k + 2) % 3 == 0)

    for t in (3, 2, 1):
        wait_scatter((NV - t) % NB)
    plsc.subcore_barrier()


def _sc_lap_body(table, edges3, zeros_slab, agg_out, ibufs, bufs, slab_sh,
                 agg_sh, semi, semg, semsc):
    c = lax.axis_index("c")
    s = lax.axis_index("s")
    arow0 = s * AGG_PER_TILE

    def run_batch(b):
        pltpu.sync_copy(zeros_slab, agg_sh.at[pl.ds(arow0, AGG_PER_TILE)])
        plsc.subcore_barrier()
        for p in range(NPASS):
            _lap_pass(table, edges3, b, p, s, ibufs, bufs, slab_sh, agg_sh,
                      semi, semg, semsc)
        pltpu.sync_copy(
            agg_sh.at[pl.ds(arow0, AGG_PER_TILE)],
            agg_out.at[b].at[pl.ds(arow0, AGG_PER_TILE)],
        )
        plsc.subcore_barrier()

    @pl.when(c == 0)
    def _():
        for b in (0, 1):
            run_batch(b)

    @pl.when(c == 1)
    def _():
        for b in (2, 3):
            run_batch(b)


@jax.jit
def _sc_lap(table, edges3, zeros_slab):
    return pl.kernel(
        _sc_lap_body,
        out_type=jax.ShapeDtypeStruct((B, NAGG, DIN), jnp.float32),
        mesh=_mesh(),
        scratch_types=[
            [pltpu.VMEM((2, LCHUNK), jnp.int32) for _ in range(NB)],
            [pltpu.VMEM((LCHUNK, DIN), jnp.float32) for _ in range(NB)],
            pltpu.VMEM_SHARED((SLAB, DIN), jnp.float32),
            pltpu.VMEM_SHARED((NAGG, DIN), jnp.float32),
            [pltpu.SemaphoreType.DMA for _ in range(NB)],
            [pltpu.SemaphoreType.DMA for _ in range(NB)],
            [pltpu.SemaphoreType.DMA for _ in range(NB)],
        ],
    )(table, edges3, zeros_slab)


_BLK = 1000


def _tc_prescale_body(x_ref, dp_ref, u0_ref, dis_ref):
    deg = dp_ref[0, :, 0] + dp_ref[1, :, 0]
    dis = jnp.where(deg > 0, lax.rsqrt(jnp.maximum(deg, 1.0)), 0.0)
    u0_ref[0] = x_ref[0] * dis[:, None]
    dis_ref[...] = jnp.broadcast_to(dis[:, None], (_BLK, 8))


@jax.jit
def _tc_prescale(x, deg_parts):
    return pl.pallas_call(
        _tc_prescale_body,
        grid=(B, N // _BLK),
        in_specs=[
            pl.BlockSpec((1, _BLK, DIN), lambda b, i: (b, i, 0)),
            pl.BlockSpec((2, _BLK, DIN), lambda b, i: (0, i, 0)),
        ],
        out_specs=[
            pl.BlockSpec((1, _BLK, DIN), lambda b, i: (b, i, 0)),
            pl.BlockSpec((_BLK, 8), lambda b, i: (i, 0)),
        ],
        out_shape=[
            jax.ShapeDtypeStruct((B, UPAD, DIN), jnp.float32),
            jax.ShapeDtypeStruct((UPAD, 8), jnp.float32),
        ],
    )(x, deg_parts)


_MBLK = NAGG // 16


def _tc_mid_body(agg_ref, dis_ref, u1_ref):
    dis = dis_ref[:, 0:1]
    u1_ref[0] = (-dis * dis) * agg_ref[0]


@jax.jit
def _tc_mid(agg1, dis):
    return pl.pallas_call(
        _tc_mid_body,
        grid=(B, 16),
        in_specs=[
            pl.BlockSpec((1, _MBLK, DIN), lambda b, i: (b, i, 0)),
            pl.BlockSpec((_MBLK, 8), lambda b, i: (i, 0)),
        ],
        out_specs=pl.BlockSpec((1, _MBLK, DIN), lambda b, i: (b, i, 0)),
        out_shape=jax.ShapeDtypeStruct((B, UPAD, DIN), jnp.float32),
    )(agg1, dis)


def _tc_final_body(x_ref, a1_ref, a2_ref, dis_ref, wc_ref, bc_ref, wl_ref,
                   bl_ref, out_ref):
    dis = dis_ref[:, 0:1]
    t0 = x_ref[0]
    t1 = -dis * a1_ref[0]
    t2 = (-2.0 * dis) * a2_ref[0] - t0
    wc = wc_ref[...]
    h = jnp.dot(t0, wc[0:DIN], preferred_element_type=jnp.float32)
    h += jnp.dot(t1, wc[DIN:2 * DIN], preferred_element_type=jnp.float32)
    h += jnp.dot(t2, wc[2 * DIN:3 * DIN], preferred_element_type=jnp.float32)
    h += bc_ref[...]
    h = jnp.maximum(h, 0.0)
    out = jnp.dot(h, wl_ref[...], preferred_element_type=jnp.float32)
    out_ref[0] = out + bl_ref[...]


@jax.jit
def _tc_final(x, agg1, agg2, dis, W_cheb, b_cheb, W_lin, b_lin):
    return pl.pallas_call(
        _tc_final_body,
        grid=(B, N // _BLK),
        in_specs=[
            pl.BlockSpec((1, _BLK, DIN), lambda b, i: (b, i, 0)),
            pl.BlockSpec((1, _BLK, DIN), lambda b, i: (b, i, 0)),
            pl.BlockSpec((1, _BLK, DIN), lambda b, i: (b, i, 0)),
            pl.BlockSpec((_BLK, 8), lambda b, i: (i, 0)),
            pl.BlockSpec((3 * DIN, DOUT), lambda b, i: (0, 0)),
            pl.BlockSpec((1, DOUT), lambda b, i: (0, 0)),
            pl.BlockSpec((DOUT, DIN), lambda b, i: (0, 0)),
            pl.BlockSpec((1, DIN), lambda b, i: (0, 0)),
        ],
        out_specs=pl.BlockSpec((1, _BLK, DIN), lambda b, i: (b, i, 0)),
        out_shape=jax.ShapeDtypeStruct((B, N, DIN), jnp.float32),
    )(x, agg1, agg2, dis, W_cheb, b_cheb.reshape(1, DOUT), W_lin,
      b_lin.reshape(1, DIN))


@jax.jit
def kernel(x, edge_index, W_cheb, b_cheb, W_lin, b_lin):
    src = edge_index[0]
    dst = edge_index[1]
    dst_deg = jnp.concatenate(
        [dst, jnp.full((E_DEG - E,), N, jnp.int32)])
    dsts32 = dst_deg.reshape(2, 16, DEG_CHUNKS, CHUNK)

    src_p = jnp.concatenate([src, jnp.zeros((E_LAP - E,), jnp.int32)])
    dst_p = jnp.concatenate([dst, jnp.full((E_LAP - E,), N, jnp.int32)])

    srcT = src_p.reshape(16, EPT)
    dstT = dst_p.reshape(16, EPT)
    srcH, srcS = srcT[:, :EH_T], srcT[:, EH_T:]
    dstH, dstS = dstT[:, :EH_T], dstT[:, EH_T:]
    dumpS = DUMP0 + (jnp.arange(16 * ES_T, dtype=jnp.int32)
                     % (NAGG - DUMP0)).reshape(16, ES_T)
    ehp = EH_T // NPASS
    passes = []
    for p in range(NPASS):
        hC = jnp.stack(
            [srcH[:, p * ehp:(p + 1) * ehp].reshape(16, NH, LCHUNK),
             dstH[:, p * ehp:(p + 1) * ehp].reshape(16, NH, LCHUNK)],
            axis=2).reshape(16, NH, 1, 2, LCHUNK)
        base = p * SLAB
        in_p = (srcS >= base) & (srcS < base + SLAB)
        g_p = jnp.where(in_p, srcS - base, 0)
        s_p = jnp.where(in_p, dstS, dumpS)
        sC = jnp.stack(
            [g_p.reshape(16, NS, LCHUNK),
             s_p.reshape(16, NS, LCHUNK)], axis=2
        ).reshape(16, NH, 2, 2, LCHUNK)
        passes.append(
            jnp.concatenate([hC, sC], axis=2).reshape(16, NV, 2, LCHUNK))
    edges3 = jnp.stack(passes)

    ones_pat = jnp.ones((CHUNK, DIN), jnp.float32)
    zeros_deg = jnp.zeros((ROWS_PER_TILE, DIN), jnp.float32)
    zeros_agg = jnp.zeros((AGG_PER_TILE, DIN), jnp.float32)

    deg_parts = _sc_deg(dsts32, ones_pat, zeros_deg)
    u0, dis = _tc_prescale(x, deg_parts)
    agg1 = _sc_lap(u0, edges3, zeros_agg)
    u1 = _tc_mid(agg1, dis)
    agg2 = _sc_lap(u1, edges3, zeros_agg)
    return _tc_final(x, agg1[:, :N], agg2[:, :N], dis, W_cheb, b_cheb,
                     W_lin, b_lin)


# speedup vs baseline: 1.5766x; 1.5766x over previous
import functools

import jax
import jax.numpy as jnp
from jax import lax
from jax.experimental import pallas as pl
from jax.experimental.pallas import tpu as pltpu
from jax.experimental.pallas import tpu_sc as plsc

B = 4
N = 10000
E = 320000
DIN = 128
DOUT = 128

NPAD = 10240
ROWS_PER_TILE = NPAD // 16
CHUNK = 128
LCHUNK = 128
NB = 2
EPT = 20480
NCHUNKS = EPT // LCHUNK
E_PAD = EPT * 16
DEG_CHUNKS = E_PAD // 32 // CHUNK

@functools.cache
def _mesh():
    return plsc.VectorSubcoreMesh(core_axis_name="c", subcore_axis_name="s")


def _sc_deg_body(dsts, ones_pat, zeros_slab, deg_out, dst_v, ones_v, deg_sh,
                 sem):
    c = lax.axis_index("c")
    s = lax.axis_index("s")
    pltpu.sync_copy(dsts.at[c].at[s], dst_v)
    pltpu.sync_copy(ones_pat, ones_v)
    pltpu.sync_copy(
        zeros_slab, deg_sh.at[pl.ds(s * ROWS_PER_TILE, ROWS_PER_TILE)])
    plsc.subcore_barrier()

    @pl.loop(0, DEG_CHUNKS)
    def _(j):
        pltpu.sync_copy(ones_v, deg_sh.at[dst_v.at[j]], add=True)

    plsc.subcore_barrier()
    pltpu.sync_copy(
        deg_sh.at[pl.ds(s * ROWS_PER_TILE, ROWS_PER_TILE)],
        deg_out.at[c].at[pl.ds(s * ROWS_PER_TILE, ROWS_PER_TILE)],
    )


@jax.jit
def _sc_deg(dsts32, ones_pat, zeros_slab):
    return pl.kernel(
        _sc_deg_body,
        out_type=jax.ShapeDtypeStruct((2, NPAD, DIN), jnp.float32),
        mesh=_mesh(),
        scratch_types=[
            pltpu.VMEM((DEG_CHUNKS, CHUNK), jnp.int32),
            pltpu.VMEM((CHUNK, DIN), jnp.float32),
            pltpu.VMEM_SHARED((NPAD, DIN), jnp.float32),
            pltpu.SemaphoreType.DMA,
        ],
    )(dsts32, ones_pat, zeros_slab)


def _lap_batch(table, edges, agg_out, b, s, ibufs, bufs, agg_sh,
               semi, semg, semsc):
    row0 = s * ROWS_PER_TILE
    tab = table.at[b]
    eds = edges.at[s]

    def wait_idx(k):
        pltpu.make_async_copy(eds.at[0], ibufs[k], semi[k]).wait()

    def fire_gather(k):
        pltpu.async_copy(tab.at[ibufs[k].at[0]], bufs[k], semg[k])

    def wait_gather(k):
        pltpu.make_async_copy(tab.at[ibufs[k].at[0]], bufs[k], semg[k]).wait()

    def sync_scatter(k):
        pltpu.sync_copy(bufs[k], agg_sh.at[ibufs[k].at[1]], add=True)

    pltpu.async_copy(eds.at[0], ibufs[0], semi[0])
    pltpu.async_copy(eds.at[1], ibufs[1], semi[1])
    wait_idx(0)
    fire_gather(0)

    @pl.loop(0, NCHUNKS // 2)
    def _(i):
        j0 = 2 * i
        wait_idx(1)
        fire_gather(1)
        wait_gather(0)
        sync_scatter(0)

        @pl.when(j0 + 2 < NCHUNKS)
        def _():
            pltpu.async_copy(eds.at[j0 + 2], ibufs[0], semi[0])
            wait_idx(0)
            fire_gather(0)

        wait_gather(1)
        sync_scatter(1)

        @pl.when(j0 + 3 < NCHUNKS)
        def _():
            pltpu.async_copy(eds.at[j0 + 3], ibufs[1], semi[1])

    plsc.subcore_barrier()
    pltpu.sync_copy(
        agg_sh.at[pl.ds(row0, ROWS_PER_TILE)],
        agg_out.at[b].at[pl.ds(row0, ROWS_PER_TILE)],
    )
    plsc.subcore_barrier()


def _lap_zero(zeros_slab, agg_sh, s):
    pltpu.sync_copy(
        zeros_slab, agg_sh.at[pl.ds(s * ROWS_PER_TILE, ROWS_PER_TILE)])
    plsc.subcore_barrier()


def _sc_lap_body(table, edges, zeros_slab, agg_out, ibufs, bufs, agg_sh,
                 semi, semg, semsc):
    c = lax.axis_index("c")
    s = lax.axis_index("s")

    @pl.when(c == 0)
    def _():
        for b in (0, 1):
            _lap_zero(zeros_slab, agg_sh, s)
            _lap_batch(table, edges, agg_out, b, s, ibufs, bufs, agg_sh,
                       semi, semg, semsc)

    @pl.when(c == 1)
    def _():
        for b in (2, 3):
            _lap_zero(zeros_slab, agg_sh, s)
            _lap_batch(table, edges, agg_out, b, s, ibufs, bufs, agg_sh,
                       semi, semg, semsc)


@jax.jit
def _sc_lap(table, edges, zeros_slab):
    return pl.kernel(
        _sc_lap_body,
        out_type=jax.ShapeDtypeStruct((B, NPAD, DIN), jnp.float32),
        mesh=_mesh(),
        scratch_types=[
            [pltpu.VMEM((2, LCHUNK), jnp.int32) for _ in range(NB)],
            [pltpu.VMEM((LCHUNK, DIN), jnp.float32) for _ in range(NB)],
            pltpu.VMEM_SHARED((NPAD, DIN), jnp.float32),
            [pltpu.SemaphoreType.DMA for _ in range(NB)],
            [pltpu.SemaphoreType.DMA for _ in range(NB)],
            [pltpu.SemaphoreType.DMA for _ in range(NB)],
        ],
    )(table, edges, zeros_slab)


_BLK = 1000


def _tc_prescale_body(x_ref, dp_ref, u0_ref, dis_ref):
    deg = dp_ref[0, :, 0] + dp_ref[1, :, 0]
    dis = jnp.where(deg > 0, lax.rsqrt(jnp.maximum(deg, 1.0)), 0.0)
    u0_ref[0] = x_ref[0] * dis[:, None]
    dis_ref[...] = jnp.broadcast_to(dis[:, None], (_BLK, 8))


@jax.jit
def _tc_prescale(x, deg_parts):
    return pl.pallas_call(
        _tc_prescale_body,
        grid=(B, N // _BLK),
        in_specs=[
            pl.BlockSpec((1, _BLK, DIN), lambda b, i: (b, i, 0)),
            pl.BlockSpec((2, _BLK, DIN), lambda b, i: (0, i, 0)),
        ],
        out_specs=[
            pl.BlockSpec((1, _BLK, DIN), lambda b, i: (b, i, 0)),
            pl.BlockSpec((_BLK, 8), lambda b, i: (i, 0)),
        ],
        out_shape=[
            jax.ShapeDtypeStruct((B, N, DIN), jnp.float32),
            jax.ShapeDtypeStruct((N, 8), jnp.float32),
        ],
    )(x, deg_parts)


def _tc_mid_body(agg_ref, dis_ref, u1_ref):
    dis = dis_ref[:, 0:1]
    u1_ref[0] = (-dis * dis) * agg_ref[0]


@jax.jit
def _tc_mid(agg1, dis):
    return pl.pallas_call(
        _tc_mid_body,
        grid=(B, N // _BLK),
        in_specs=[
            pl.BlockSpec((1, _BLK, DIN), lambda b, i: (b, i, 0)),
            pl.BlockSpec((_BLK, 8), lambda b, i: (i, 0)),
        ],
        out_specs=pl.BlockSpec((1, _BLK, DIN), lambda b, i: (b, i, 0)),
        out_shape=jax.ShapeDtypeStruct((B, N, DIN), jnp.float32),
    )(agg1, dis)


def _tc_final_body(x_ref, a1_ref, a2_ref, dis_ref, wc_ref, bc_ref, wl_ref,
                   bl_ref, out_ref):
    dis = dis_ref[:, 0:1]
    t0 = x_ref[0]
    t1 = -dis * a1_ref[0]
    t2 = (-2.0 * dis) * a2_ref[0] - t0
    wc = wc_ref[...]
    h = jnp.dot(t0, wc[0:DIN], preferred_element_type=jnp.float32)
    h += jnp.dot(t1, wc[DIN:2 * DIN], preferred_element_type=jnp.float32)
    h += jnp.dot(t2, wc[2 * DIN:3 * DIN], preferred_element_type=jnp.float32)
    h += bc_ref[...]
    h = jnp.maximum(h, 0.0)
    out = jnp.dot(h, wl_ref[...], preferred_element_type=jnp.float32)
    out_ref[0] = out + bl_ref[...]


@jax.jit
def _tc_final(x, agg1, agg2, dis, W_cheb, b_cheb, W_lin, b_lin):
    return pl.pallas_call(
        _tc_final_body,
        grid=(B, N // _BLK),
        in_specs=[
            pl.BlockSpec((1, _BLK, DIN), lambda b, i: (b, i, 0)),
            pl.BlockSpec((1, _BLK, DIN), lambda b, i: (b, i, 0)),
            pl.BlockSpec((1, _BLK, DIN), lambda b, i: (b, i, 0)),
            pl.BlockSpec((_BLK, 8), lambda b, i: (i, 0)),
            pl.BlockSpec((3 * DIN, DOUT), lambda b, i: (0, 0)),
            pl.BlockSpec((1, DOUT), lambda b, i: (0, 0)),
            pl.BlockSpec((DOUT, DIN), lambda b, i: (0, 0)),
            pl.BlockSpec((1, DIN), lambda b, i: (0, 0)),
        ],
        out_specs=pl.BlockSpec((1, _BLK, DIN), lambda b, i: (b, i, 0)),
        out_shape=jax.ShapeDtypeStruct((B, N, DIN), jnp.float32),
    )(x, agg1, agg2, dis, W_cheb, b_cheb.reshape(1, DOUT), W_lin,
      b_lin.reshape(1, DIN))


@jax.jit
def kernel(x, edge_index, W_cheb, b_cheb, W_lin, b_lin):
    src = edge_index[0]
    dst = edge_index[1]
    pad = E_PAD - E
    src_p = jnp.concatenate([src, jnp.zeros((pad,), jnp.int32)])
    dst_p = jnp.concatenate([dst, jnp.full((pad,), N, jnp.int32)])
    srcs = src_p.reshape(16, NCHUNKS, 1, LCHUNK)
    dsts = dst_p.reshape(16, NCHUNKS, 1, LCHUNK)
    edges = jnp.concatenate([srcs, dsts], axis=2)
    dsts32 = dst_p.reshape(2, 16, DEG_CHUNKS, CHUNK)

    ones_pat = jnp.ones((CHUNK, DIN), jnp.float32)
    zeros_slab = jnp.zeros((ROWS_PER_TILE, DIN), jnp.float32)

    deg_parts = _sc_deg(dsts32, ones_pat, zeros_slab)
    u0, dis = _tc_prescale(x, deg_parts)
    agg1 = _sc_lap(u0, edges, zeros_slab)
    u1 = _tc_mid(agg1[:, :N], dis)
    agg2 = _sc_lap(u1, edges, zeros_slab)
    return _tc_final(x, agg1[:, :N], agg2[:, :N], dis, W_cheb, b_cheb,
                     W_lin, b_lin)
